# Initial kernel scaffold; baseline (speedup 1.0000x reference)
import jax
import jax.numpy as jnp
from jax.experimental import pallas as pl


def kernel(x, edge_index, edge_label_index, batch, Ws1, Wn1, b1, Ws2, Wn2, b2, Wc, bc):
    raise NotImplementedError("write your pallas kernel here")


# speedup vs baseline: 4.1787x; 4.1787x over previous
import functools

import jax
import jax.numpy as jnp
from jax import lax
from jax.experimental import pallas as pl
from jax.experimental.pallas import tpu as pltpu
from jax.experimental.pallas import tpu_sc as plsc

N = 10000
D = 128
E = 320000
EL = 100000

NC = 2
NS = 16
CH = 128
CH_PER_TILE = 80
CH_PER_CORE = CH_PER_TILE * NS
CH_TOTAL = CH_PER_CORE * NC
E_PAD = CH_TOTAL * CH - E
N_DUMP = 112
N_ACC = N + N_DUMP
ZR_PER_TILE = N_ACC // NS
RP = 624
R_TAIL = N - RP * NS

ELC = 25
ELP_PER_TILE = ELC * 128
ELP = ELP_PER_TILE * NC * NS

_mesh = lambda: plsc.VectorSubcoreMesh(core_axis_name="c", subcore_axis_name="s")


def _acc_to_out(acc_sh, out_hbm, s):
    ro = s * RP
    pltpu.sync_copy(acc_sh.at[pl.ds(ro, RP)], out_hbm.at[pl.ds(ro, RP)])

    @pl.when(s == NS - 1)
    def _tail():
        pltpu.sync_copy(acc_sh.at[pl.ds(RP * NS, R_TAIL)],
                        out_hbm.at[pl.ds(RP * NS, R_TAIL)])


CH_ROUND = CH_PER_TILE // 2


@functools.partial(
    pl.kernel, mesh=_mesh(),
    out_type=jax.ShapeDtypeStruct((NC, N, D), jnp.float32),
    scratch_types=[
        pltpu.VMEM((CH_ROUND, CH), jnp.int32),
        pltpu.VMEM((CH_ROUND, CH), jnp.int32),
        pltpu.VMEM((CH, D), jnp.float32),
        pltpu.VMEM((CH, D), jnp.float32),
        pltpu.VMEM_SHARED((N_ACC, D), jnp.float32),
        pltpu.SemaphoreType.DMA,
        pltpu.SemaphoreType.DMA,
    ],
)
def _seg_pass(src_hbm, dst_hbm, x_hbm, z128_hbm,
              agg_out,
              srcv, dstv, rows0, rows1, acc_sh, sem0, sem1):
    c = lax.axis_index("c")
    s = lax.axis_index("s")
    rz = s * ZR_PER_TILE
    pltpu.sync_copy(z128_hbm.at[pl.ds(rz, ZR_PER_TILE)],
                    acc_sh.at[pl.ds(rz, ZR_PER_TILE)])
    base_chunk = c * CH_PER_CORE + s * CH_PER_TILE
    plsc.subcore_barrier()

    for r in range(CH_PER_TILE // CH_ROUND):
        pltpu.sync_copy(
            src_hbm.at[pl.ds(base_chunk + r * CH_ROUND, CH_ROUND)], srcv)
        pltpu.sync_copy(
            dst_hbm.at[pl.ds(base_chunk + r * CH_ROUND, CH_ROUND)], dstv)
        pltpu.async_copy(x_hbm.at[srcv.at[0]], rows0, sem0)

        def body(t, carry):
            j = 2 * t
            pltpu.async_copy(x_hbm.at[srcv.at[j + 1]], rows1, sem1)
            pltpu.make_async_copy(x_hbm.at[pl.ds(0, CH)], rows0, sem0).wait()
            pltpu.sync_copy(rows0, acc_sh.at[dstv.at[j]], add=True)

            @pl.when(j + 2 < CH_ROUND)
            def _nx():
                pltpu.async_copy(x_hbm.at[srcv.at[j + 2]], rows0, sem0)

            pltpu.make_async_copy(x_hbm.at[pl.ds(0, CH)], rows1, sem1).wait()
            pltpu.sync_copy(rows1, acc_sh.at[dstv.at[j + 1]], add=True)
            return carry

        lax.fori_loop(0, CH_ROUND // 2, body, 0)

    plsc.subcore_barrier()
    _acc_to_out(acc_sh, agg_out.at[c], s)


@functools.partial(
    pl.kernel, mesh=_mesh(),
    out_type=jax.ShapeDtypeStruct((NC * N,), jnp.float32),
    scratch_types=[
        pltpu.VMEM((CH_PER_TILE, CH), jnp.int32),
        pltpu.VMEM((CH,), jnp.int32),
        pltpu.VMEM((CH,), jnp.float32),
        pltpu.VMEM((ZR_PER_TILE,), jnp.float32),
        pltpu.VMEM((RP,), jnp.float32),
        pltpu.VMEM_SHARED((N_ACC,), jnp.float32),
    ],
)
def _deg_pass(dst_hbm, deg_out, dstv, idxc, onesv, zb, ob, deg_sh):
    c = lax.axis_index("c")
    s = lax.axis_index("s")
    for k in range(CH // 16):
        onesv[pl.ds(k * 16, 16)] = jnp.full((16,), 1.0, jnp.float32)
    for k in range(ZR_PER_TILE // 16):
        zb[pl.ds(k * 16, 16)] = jnp.zeros((16,), jnp.float32)
    zb[pl.ds(ZR_PER_TILE - 16, 16)] = jnp.zeros((16,), jnp.float32)
    rz = s * ZR_PER_TILE
    pltpu.sync_copy(zb, deg_sh.at[pl.ds(rz, ZR_PER_TILE)])
    base_chunk = c * CH_PER_CORE + s * CH_PER_TILE
    pltpu.sync_copy(dst_hbm.at[pl.ds(base_chunk, CH_PER_TILE)], dstv)
    plsc.subcore_barrier()

    def body(j, carry):
        for k in range(CH // 16):
            idxc[pl.ds(k * 16, 16)] = dstv[j, pl.ds(k * 16, 16)]
        pltpu.sync_copy(onesv, deg_sh.at[idxc], add=True)
        return carry

    lax.fori_loop(0, CH_PER_TILE, body, 0)
    plsc.subcore_barrier()
    ro = s * RP
    pltpu.sync_copy(deg_sh.at[pl.ds(ro, RP)], ob)
    pltpu.sync_copy(ob, deg_out.at[pl.ds(c * N + ro, RP)])

    @pl.when(s == NS - 1)
    def _t():
        pltpu.sync_copy(deg_sh.at[pl.ds(RP * NS, R_TAIL)],
                        ob.at[pl.ds(0, R_TAIL)])
        pltpu.sync_copy(ob.at[pl.ds(0, R_TAIL)],
                        deg_out.at[pl.ds(c * N + RP * NS, R_TAIL)])


@functools.partial(
    pl.kernel, mesh=_mesh(),
    out_type=jax.ShapeDtypeStruct((NC * NS, ELC, 128), jnp.float32),
    scratch_types=[
        pltpu.VMEM((ELC, 128), jnp.int32),
        pltpu.VMEM((ELC, 128), jnp.int32),
        pltpu.VMEM((ELC, 128), jnp.float32),
        pltpu.VMEM((ELC, 128), jnp.float32),
        pltpu.VMEM((ELC, 128), jnp.float32),
        pltpu.VMEM((N,), jnp.float32),
        pltpu.VMEM_SHARED((N,), jnp.float32),
        pltpu.SemaphoreType.DMA,
    ],
)
def _head(s_hbm, el0_hbm, el1_hbm, out_hbm, i0v, i1v, v0, v1, ov, sbuf, s_sp,
          sem):
    c = lax.axis_index("c")
    s = lax.axis_index("s")
    w = s * NC + c
    @pl.when(s == 0)
    def _stage():
        pltpu.sync_copy(s_hbm, sbuf)
        pltpu.sync_copy(sbuf, s_sp)

    pltpu.sync_copy(el0_hbm.at[w], i0v)
    pltpu.sync_copy(el1_hbm.at[w], i1v)
    plsc.subcore_barrier()

    def fire(j, carry):
        pltpu.async_copy(s_sp.at[i0v.at[j]], v0.at[j], sem)
        pltpu.async_copy(s_sp.at[i1v.at[j]], v1.at[j], sem)
        return carry

    lax.fori_loop(0, ELC, fire, 0)
    pltpu.make_async_copy(out_hbm.at[w], v0, sem).wait()
    pltpu.make_async_copy(out_hbm.at[w], v1, sem).wait()

    def add_body(j, carry):
        for k in range(8):
            ov[j, pl.ds(k * 16, 16)] = (v0[j, pl.ds(k * 16, 16)] +
                                        v1[j, pl.ds(k * 16, 16)])
        return carry

    lax.fori_loop(0, ELC, add_body, 0)
    pltpu.sync_copy(ov, out_hbm.at[w])


BN = 2000


def _tc1_body(x_ref, agg_ref, d0_ref, d1_ref, ws_ref, wn_ref, b_ref, o_ref):
    deg = jnp.maximum(d0_ref[...] + d1_ref[...], 1.0)
    m = (agg_ref[0] + agg_ref[1]) / deg
    h = jnp.dot(x_ref[...], ws_ref[...], preferred_element_type=jnp.float32)
    h = h + jnp.dot(m, wn_ref[...], preferred_element_type=jnp.float32)
    h = h + b_ref[...]
    o_ref[...] = jnp.maximum(h, 0.0)


def _tc2_body(h_ref, agg_ref, d0_ref, d1_ref, ws_ref, wn_ref, b_ref, wc_ref,
              bc_ref, o_ref):
    u = jnp.dot(ws_ref[...], wc_ref[...], preferred_element_type=jnp.float32)
    v = jnp.dot(wn_ref[...], wc_ref[...], preferred_element_type=jnp.float32)
    deg = jnp.maximum(d0_ref[...] + d1_ref[...], 1.0)
    m = (agg_ref[0] + agg_ref[1]) / deg
    sv = jnp.dot(h_ref[...], u, preferred_element_type=jnp.float32)
    sv = sv + jnp.dot(m, v, preferred_element_type=jnp.float32)
    const = jnp.dot(b_ref[...], wc_ref[...], preferred_element_type=jnp.float32)
    o_ref[...] = sv + const + 0.5 * bc_ref[...]


def _tc1(x, agg, d0, d1, Ws1, Wn1, b1):
    return pl.pallas_call(
        _tc1_body,
        grid=(N // BN,),
        in_specs=[
            pl.BlockSpec((BN, D), lambda i: (i, 0)),
            pl.BlockSpec((NC, BN, D), lambda i: (0, i, 0)),
            pl.BlockSpec((BN, 1), lambda i: (i, 0)),
            pl.BlockSpec((BN, 1), lambda i: (i, 0)),
            pl.BlockSpec((D, D), lambda i: (0, 0)),
            pl.BlockSpec((D, D), lambda i: (0, 0)),
            pl.BlockSpec((1, D), lambda i: (0, 0)),
        ],
        out_specs=pl.BlockSpec((BN, D), lambda i: (i, 0)),
        out_shape=jax.ShapeDtypeStruct((N, D), jnp.float32),
    )(x, agg, d0, d1, Ws1, Wn1, b1)


def _tc2(h1, agg, d0, d1, Ws2, Wn2, b2, Wc, bc):
    return pl.pallas_call(
        _tc2_body,
        grid=(N // BN,),
        in_specs=[
            pl.BlockSpec((BN, D), lambda i: (i, 0)),
            pl.BlockSpec((NC, BN, D), lambda i: (0, i, 0)),
            pl.BlockSpec((BN, 1), lambda i: (i, 0)),
            pl.BlockSpec((BN, 1), lambda i: (i, 0)),
            pl.BlockSpec((D, D), lambda i: (0, 0)),
            pl.BlockSpec((D, D), lambda i: (0, 0)),
            pl.BlockSpec((1, D), lambda i: (0, 0)),
            pl.BlockSpec((D, 1), lambda i: (0, 0)),
            pl.BlockSpec((1, 1), lambda i: (0, 0)),
        ],
        out_specs=pl.BlockSpec((BN, 1), lambda i: (i, 0)),
        out_shape=jax.ShapeDtypeStruct((N, 1), jnp.float32),
    )(h1, agg, d0, d1, Ws2, Wn2, b2, Wc, bc)


def kernel(x, edge_index, edge_label_index, batch,
           Ws1, Wn1, b1, Ws2, Wn2, b2, Wc, bc):
    del batch
    pad_src = jnp.zeros((E_PAD,), jnp.int32)
    pad_dst = N + (jnp.arange(E_PAD, dtype=jnp.int32) % N_DUMP)
    src2d = jnp.concatenate([edge_index[0], pad_src]).reshape(CH_TOTAL, CH)
    dst2d = jnp.concatenate([edge_index[1], pad_dst]).reshape(CH_TOTAL, CH)
    z128 = jnp.zeros((N_ACC, D), jnp.float32)

    degflat = _deg_pass(dst2d)
    d0 = degflat[:N].reshape(N, 1)
    d1 = degflat[N:].reshape(N, 1)
    agg1 = _seg_pass(src2d, dst2d, x, z128)
    h1 = _tc1(x, agg1, d0, d1, Ws1, Wn1, b1.reshape(1, D))
    agg2 = _seg_pass(src2d, dst2d, h1, z128)
    s = _tc2(h1, agg2, d0, d1, Ws2, Wn2, b2.reshape(1, D), Wc,
             bc.reshape(1, 1))

    padl = jnp.zeros((ELP - EL,), jnp.int32)
    el0 = jnp.concatenate([edge_label_index[0], padl]).reshape(NC * NS, ELC,
                                                                128)
    el1 = jnp.concatenate([edge_label_index[1], padl]).reshape(NC * NS, ELC,
                                                                128)
    out = _head(s.reshape(N), el0, el1)
    return out.reshape(ELP)[:EL].reshape(EL, 1)
